# baseline (speedup 1.0000x reference)
import functools

import jax
import jax.numpy as jnp
from jax import lax
from jax.experimental import pallas as pl
from jax.experimental.pallas import tpu as pltpu
from jax.experimental.pallas import tpu_sc as plsc

NUM = 16384
C = 1000
C_BUF = C
C_PAD = 1024

NC = 2
NS = 16
L = 16
NW = NC * NS
BPW = NUM // NW
RBLK = 32
NBLK = BPW // RBLK
PACK = 128 // L
OUT_R = NUM // PACK


def _sc_select(inputs, targets, alpha_pad):
    mesh = plsc.VectorSubcoreMesh(core_axis_name="c", subcore_axis_name="s")

    @functools.partial(
        pl.kernel,
        mesh=mesh,
        out_type=(
            jax.ShapeDtypeStruct((OUT_R, 128), jnp.float32),
            jax.ShapeDtypeStruct((OUT_R, 128), jnp.float32),
        ),
        scratch_types=[
            pltpu.VMEM((RBLK, C), jnp.float32),
            pltpu.VMEM((BPW,), jnp.int32),
            pltpu.VMEM((BPW // PACK, 128), jnp.float32),
            pltpu.VMEM((BPW // PACK, 128), jnp.float32),
            pltpu.VMEM((C_PAD,), jnp.float32),
            pltpu.SemaphoreType.DMA,
        ],
    )
    def k(x_hbm, tgt_hbm, alpha_hbm, p_hbm, a_hbm,
          xbuf, tgt_v, p_v, a_v, al_v, sem):
        wid = lax.axis_index("s") * NC + lax.axis_index("c")
        base = wid * BPW
        pltpu.sync_copy(tgt_hbm.at[pl.ds(base, BPW)], tgt_v)
        pltpu.sync_copy(alpha_hbm, al_v)
        lane = lax.iota(jnp.int32, L)
        for b in range(NBLK):
            pltpu.sync_copy(x_hbm.at[pl.ds(base + b * RBLK, RBLK)], xbuf)
            for r in range(RBLK):
                n_loc = b * RBLK + r
                if r % L == 0:
                    tvec = tgt_v[pl.ds(n_loc, L)]
                t_r = tvec[r % L]
                c0 = pl.multiple_of((t_r >> 4) << 4, L)
                sel = lane == (t_r & 15)
                chunk = xbuf[r, pl.ds(c0, L)]
                achunk = al_v[pl.ds(c0, L)]
                pm = jnp.where(sel, chunk, 0.0)
                am = jnp.where(sel, achunk, 0.0)
                p_v[n_loc // PACK, pl.ds((n_loc % PACK) * L, L)] = pm
                a_v[n_loc // PACK, pl.ds((n_loc % PACK) * L, L)] = am
        pltpu.sync_copy(p_v, p_hbm.at[pl.ds(wid * (BPW // PACK), BPW // PACK)])
        pltpu.sync_copy(a_v, a_hbm.at[pl.ds(wid * (BPW // PACK), BPW // PACK)])

    return k(inputs, targets, alpha_pad)


def _tc_body(p_ref, a_ref, o_ref):
    x = p_ref[...]
    y = a_ref[...]
    rows = jax.lax.broadcasted_iota(jnp.int32, (128, PACK), 0)
    cols = jax.lax.broadcasted_iota(jnp.int32, (128, PACK), 1)
    seg = (rows // L == cols).astype(jnp.float32)
    p = jax.lax.dot_general(x, seg, (((1,), (0,)), ((), ())),
                            preferred_element_type=jnp.float32)
    a = jax.lax.dot_general(y, seg, (((1,), (0,)), ((), ())),
                            preferred_element_type=jnp.float32)
    om = 1.0 - p
    o_ref[0, 0] = jnp.sum((a * om * om) * jnp.log(p)) * (-1.0 / NUM)


def _tc_focal(p2d, a2d):
    out = pl.pallas_call(
        _tc_body,
        out_shape=jax.ShapeDtypeStruct((1, 1), jnp.float32),
        out_specs=pl.BlockSpec(memory_space=pltpu.SMEM),
    )(p2d, a2d)
    return out[0, 0]


def kernel(inputs, targets, alpha):
    tgt = targets.astype(jnp.int32)
    al = jnp.pad(alpha.reshape(-1), (0, C_PAD - C))
    p2d, a2d = _sc_select(inputs, tgt, al)
    return _tc_focal(p2d, a2d)


# speedup vs baseline: 1.0441x; 1.0441x over previous
import functools

import jax
import jax.numpy as jnp
from jax import lax
from jax.experimental import pallas as pl
from jax.experimental.pallas import tpu as pltpu
from jax.experimental.pallas import tpu_sc as plsc

NUM = 16384
C = 1000
C_BUF = C
C_PAD = 1024

NC = 2
NS = 16
L = 16
NW = NC * NS
BPW = NUM // NW
RBLK = 32
NBLK = BPW // RBLK
PACK = 128 // L
OUT_R = NUM // PACK


def _sc_select(inputs, targets, alpha_pad):
    mesh = plsc.VectorSubcoreMesh(core_axis_name="c", subcore_axis_name="s")

    @functools.partial(
        pl.kernel,
        mesh=mesh,
        out_type=(
            jax.ShapeDtypeStruct((OUT_R, 128), jnp.float32),
            jax.ShapeDtypeStruct((OUT_R, 128), jnp.float32),
        ),
        scratch_types=[
            pltpu.VMEM((RBLK, C), jnp.float32),
            pltpu.VMEM((RBLK, C), jnp.float32),
            pltpu.VMEM((BPW,), jnp.int32),
            pltpu.VMEM((BPW // PACK, 128), jnp.float32),
            pltpu.VMEM((BPW // PACK, 128), jnp.float32),
            pltpu.VMEM((C_PAD,), jnp.float32),
            pltpu.SemaphoreType.DMA,
            pltpu.SemaphoreType.DMA,
        ],
    )
    def k(x_hbm, tgt_hbm, alpha_hbm, p_hbm, a_hbm,
          xbuf0, xbuf1, tgt_v, p_v, a_v, al_v, sem0, sem1):
        wid = lax.axis_index("s") * NC + lax.axis_index("c")
        base = wid * BPW
        pltpu.sync_copy(tgt_hbm.at[pl.ds(base, BPW)], tgt_v)
        pltpu.sync_copy(alpha_hbm, al_v)
        lane = lax.iota(jnp.int32, L)
        bufs = (xbuf0, xbuf1)
        sems = (sem0, sem1)
        cps = [None] * NBLK
        cps[0] = pltpu.async_copy(x_hbm.at[pl.ds(base, RBLK)], bufs[0], sems[0])
        for b in range(NBLK):
            if b + 1 < NBLK:
                cps[b + 1] = pltpu.async_copy(
                    x_hbm.at[pl.ds(base + (b + 1) * RBLK, RBLK)],
                    bufs[(b + 1) % 2], sems[(b + 1) % 2])
            cps[b].wait()
            xbuf = bufs[b % 2]
            for r in range(RBLK):
                n_loc = b * RBLK + r
                if r % L == 0:
                    tvec = tgt_v[pl.ds(n_loc, L)]
                t_r = tvec[r % L]
                c0 = pl.multiple_of((t_r >> 4) << 4, L)
                sel = lane == (t_r & 15)
                chunk = xbuf[r, pl.ds(c0, L)]
                achunk = al_v[pl.ds(c0, L)]
                pm = jnp.where(sel, chunk, 0.0)
                am = jnp.where(sel, achunk, 0.0)
                p_v[n_loc // PACK, pl.ds((n_loc % PACK) * L, L)] = pm
                a_v[n_loc // PACK, pl.ds((n_loc % PACK) * L, L)] = am
        pltpu.sync_copy(p_v, p_hbm.at[pl.ds(wid * (BPW // PACK), BPW // PACK)])
        pltpu.sync_copy(a_v, a_hbm.at[pl.ds(wid * (BPW // PACK), BPW // PACK)])

    return k(inputs, targets, alpha_pad)


def _tc_body(p_ref, a_ref, o_ref):
    x = p_ref[...]
    y = a_ref[...]
    rows = jax.lax.broadcasted_iota(jnp.int32, (128, PACK), 0)
    cols = jax.lax.broadcasted_iota(jnp.int32, (128, PACK), 1)
    seg = (rows // L == cols).astype(jnp.float32)
    p = jax.lax.dot_general(x, seg, (((1,), (0,)), ((), ())),
                            preferred_element_type=jnp.float32)
    a = jax.lax.dot_general(y, seg, (((1,), (0,)), ((), ())),
                            preferred_element_type=jnp.float32)
    om = 1.0 - p
    o_ref[0, 0] = jnp.sum((a * om * om) * jnp.log(p)) * (-1.0 / NUM)


def _tc_focal(p2d, a2d):
    out = pl.pallas_call(
        _tc_body,
        out_shape=jax.ShapeDtypeStruct((1, 1), jnp.float32),
        out_specs=pl.BlockSpec(memory_space=pltpu.SMEM),
    )(p2d, a2d)
    return out[0, 0]


def kernel(inputs, targets, alpha):
    tgt = targets.astype(jnp.int32)
    al = jnp.pad(alpha.reshape(-1), (0, C_PAD - C))
    p2d, a2d = _sc_select(inputs, tgt, al)
    return _tc_focal(p2d, a2d)


# speedup vs baseline: 1.1431x; 1.0948x over previous
import functools

import jax
import jax.numpy as jnp
from jax import lax
from jax.experimental import pallas as pl
from jax.experimental.pallas import tpu as pltpu
from jax.experimental.pallas import tpu_sc as plsc

NUM = 16384
C = 1000
C_PAD = 1024

NC = 2
NS = 16
L = 16
NW = NC * NS
PACK = 128 // L

SPLIT = 8192
SC_ROWS = NUM - SPLIT
BPW = SC_ROWS // NW
RBLK = 32
NBLK = BPW // RBLK
OUT_R = SC_ROWS // PACK

TC_BLK = 2048
TC_GRID = SPLIT // TC_BLK


def _sc_select(inputs, targets_sc, alpha_pad):
    mesh = plsc.VectorSubcoreMesh(core_axis_name="c", subcore_axis_name="s")

    @functools.partial(
        pl.kernel,
        mesh=mesh,
        out_type=(
            jax.ShapeDtypeStruct((OUT_R, 128), jnp.float32),
            jax.ShapeDtypeStruct((OUT_R, 128), jnp.float32),
        ),
        scratch_types=[
            pltpu.VMEM((RBLK, C), jnp.float32),
            pltpu.VMEM((RBLK, C), jnp.float32),
            pltpu.VMEM((BPW,), jnp.int32),
            pltpu.VMEM((BPW // PACK, 128), jnp.float32),
            pltpu.VMEM((BPW // PACK, 128), jnp.float32),
            pltpu.VMEM((C_PAD,), jnp.float32),
            pltpu.SemaphoreType.DMA,
            pltpu.SemaphoreType.DMA,
        ],
    )
    def k(x_hbm, tgt_hbm, alpha_hbm, p_hbm, a_hbm,
          xbuf0, xbuf1, tgt_v, p_v, a_v, al_v, sem0, sem1):
        wid = lax.axis_index("s") * NC + lax.axis_index("c")
        base = wid * BPW
        pltpu.sync_copy(tgt_hbm.at[pl.ds(base, BPW)], tgt_v)
        pltpu.sync_copy(alpha_hbm, al_v)
        lane = lax.iota(jnp.int32, L)
        bufs = (xbuf0, xbuf1)
        sems = (sem0, sem1)
        cps = [None] * NBLK
        cps[0] = pltpu.async_copy(
            x_hbm.at[pl.ds(SPLIT + base, RBLK)], bufs[0], sems[0])
        for b in range(NBLK):
            if b + 1 < NBLK:
                cps[b + 1] = pltpu.async_copy(
                    x_hbm.at[pl.ds(SPLIT + base + (b + 1) * RBLK, RBLK)],
                    bufs[(b + 1) % 2], sems[(b + 1) % 2])
            cps[b].wait()
            xbuf = bufs[b % 2]
            for r in range(RBLK):
                n_loc = b * RBLK + r
                if r % L == 0:
                    tvec = tgt_v[pl.ds(n_loc, L)]
                t_r = tvec[r % L]
                c0 = pl.multiple_of((t_r >> 4) << 4, L)
                sel = lane == (t_r & 15)
                chunk = xbuf[r, pl.ds(c0, L)]
                achunk = al_v[pl.ds(c0, L)]
                pm = jnp.where(sel, chunk, 0.0)
                am = jnp.where(sel, achunk, 0.0)
                p_v[n_loc // PACK, pl.ds((n_loc % PACK) * L, L)] = pm
                a_v[n_loc // PACK, pl.ds((n_loc % PACK) * L, L)] = am
        pltpu.sync_copy(p_v, p_hbm.at[pl.ds(wid * (BPW // PACK), BPW // PACK)])
        pltpu.sync_copy(a_v, a_hbm.at[pl.ds(wid * (BPW // PACK), BPW // PACK)])

    return k(inputs, targets_sc, alpha_pad)


def _tc_main_body(t_ref, al_ref, x_ref, o_ref):
    i = pl.program_id(0)
    x = x_ref[...]
    t = t_ref[0, 0, :]
    cols = jax.lax.broadcasted_iota(jnp.int32, (TC_BLK, C), 1)
    mask = (cols == t[:, None]).astype(jnp.float32)
    p = jnp.sum(x * mask, axis=1)
    a = jax.lax.dot_general(mask, al_ref[...], (((1,), (1,)), ((), ())),
                            preferred_element_type=jnp.float32)[:, 0]
    om = 1.0 - p
    part = jnp.sum((a * om * om) * jnp.log(p))

    @pl.when(i == 0)
    def _():
        o_ref[0, 0] = 0.0

    o_ref[0, 0] += part


def _tc_main(tgt_tc, al2d, inputs):
    t3d = tgt_tc.reshape(TC_GRID, 1, TC_BLK)
    return pl.pallas_call(
        _tc_main_body,
        grid=(TC_GRID,),
        in_specs=[
            pl.BlockSpec((1, 1, TC_BLK), lambda i: (i, 0, 0)),
            pl.BlockSpec((1, C), lambda i: (0, 0)),
            pl.BlockSpec((TC_BLK, C), lambda i: (i, 0)),
        ],
        out_specs=pl.BlockSpec(memory_space=pltpu.SMEM),
        out_shape=jax.ShapeDtypeStruct((1, 1), jnp.float32),
    )(t3d, al2d, inputs)


def _tc_fin_body(p_ref, a_ref, part_ref, o_ref):
    x = p_ref[...]
    y = a_ref[...]
    rows = jax.lax.broadcasted_iota(jnp.int32, (128, PACK), 0)
    cols = jax.lax.broadcasted_iota(jnp.int32, (128, PACK), 1)
    seg = (rows // L == cols).astype(jnp.float32)
    p = jax.lax.dot_general(x, seg, (((1,), (0,)), ((), ())),
                            preferred_element_type=jnp.float32)
    a = jax.lax.dot_general(y, seg, (((1,), (0,)), ((), ())),
                            preferred_element_type=jnp.float32)
    om = 1.0 - p
    sc_sum = jnp.sum((a * om * om) * jnp.log(p))
    o_ref[0, 0] = (sc_sum + part_ref[0, 0]) * (-1.0 / NUM)


def _tc_finish(p2d, a2d, partial):
    out = pl.pallas_call(
        _tc_fin_body,
        in_specs=[
            pl.BlockSpec((OUT_R, 128), lambda: (0, 0)),
            pl.BlockSpec((OUT_R, 128), lambda: (0, 0)),
            pl.BlockSpec(memory_space=pltpu.SMEM),
        ],
        out_specs=pl.BlockSpec(memory_space=pltpu.SMEM),
        out_shape=jax.ShapeDtypeStruct((1, 1), jnp.float32),
    )(p2d, a2d, partial)
    return out[0, 0]


def kernel(inputs, targets, alpha):
    tgt = targets.astype(jnp.int32)
    al = jnp.pad(alpha.reshape(-1), (0, C_PAD - C))
    al2d = alpha.reshape(1, C)
    p2d, a2d = _sc_select(inputs, tgt[SPLIT:], al)
    partial = _tc_main(tgt[:SPLIT], al2d, inputs)
    return _tc_finish(p2d, a2d, partial)
